# baseline (speedup 1.0000x reference)
import functools

import jax
import jax.numpy as jnp
from jax import lax
from jax.experimental import pallas as pl
from jax.experimental.pallas import tpu as pltpu
from jax.experimental.pallas import tpu_sc as plsc

M = 6
K = 64
NC = 2
NS = 16
NW = NC * NS
L = 16

CHUNK = 4096
NBUF = 2
UNROLL = 8


@functools.lru_cache(maxsize=None)
def _build(batch: int):
    assert batch % (NW * CHUNK) == 0
    rows_per_worker = batch // NW
    nchunks = rows_per_worker // CHUNK

    mesh = plsc.VectorSubcoreMesh(
        core_axis_name="c", subcore_axis_name="s",
        num_cores=NC, num_subcores=NS,
    )

    @functools.partial(
        pl.kernel,
        out_type=jax.ShapeDtypeStruct((batch,), jnp.float32),
        mesh=mesh,
        scratch_types=[
            pltpu.VMEM((CHUNK * M,), jnp.int32),
            pltpu.VMEM((CHUNK * M,), jnp.int32),
            pltpu.VMEM((K,), jnp.float32),
            pltpu.VMEM((CHUNK,), jnp.float32),
            pltpu.VMEM((CHUNK,), jnp.float32),
            pltpu.SemaphoreType.DMA,
            pltpu.SemaphoreType.DMA,
            pltpu.SemaphoreType.DMA,
            pltpu.SemaphoreType.DMA,
        ],
        compiler_params=pltpu.CompilerParams(needs_layout_passes=False),
    )
    def mapper(b_hbm, sym_hbm, out_hbm, b_v0, b_v1, sym_v, o_v0, o_v1,
               si0, si1, so0, so1):
        b_bufs = (b_v0, b_v1)
        o_bufs = (o_v0, o_v1)
        sems_in = (si0, si1)
        sems_out = (so0, so1)

        wid = lax.axis_index("s") * NC + lax.axis_index("c")
        row0 = wid * rows_per_worker
        pltpu.sync_copy(sym_hbm, sym_v)
        lane6 = lax.iota(jnp.int32, L) * M

        def in_copy(c):
            buf = c % NBUF
            return pltpu.make_async_copy(
                b_hbm.at[pl.ds((row0 + c * CHUNK) * M, CHUNK * M)],
                b_bufs[buf], sems_in[buf])

        def out_copy(c):
            buf = c % NBUF
            return pltpu.make_async_copy(
                o_bufs[buf],
                out_hbm.at[pl.ds(row0 + c * CHUNK, CHUNK)],
                sems_out[buf])

        for c in range(min(NBUF, nchunks)):
            in_copy(c).start()

        for c in range(nchunks):
            buf = c % NBUF
            in_copy(c).wait()
            if c >= NBUF:
                out_copy(c - NBUF).wait()

            bbuf = b_bufs[buf]
            obuf = o_bufs[buf]

            @plsc.parallel_loop(0, CHUNK // L, unroll=UNROLL)
            def _(g):
                base = lane6 + g * (L * M)
                acc = plsc.load_gather(bbuf, [base])
                for j in range(1, M):
                    acc = acc * 2 + plsc.load_gather(bbuf, [base + j])
                obuf[pl.ds(g * L, L)] = plsc.load_gather(sym_v, [acc])

            out_copy(c).start()
            if c + NBUF < nchunks:
                in_copy(c + NBUF).start()

        for c in range(max(nchunks - NBUF, 0), nchunks):
            out_copy(c).wait()

    return mapper


def kernel(b, symbols):
    batch = b.shape[0]
    flat = _build(batch)(b.reshape(-1), symbols.reshape(-1))
    return flat.reshape(batch, 1, 1)


# speedup vs baseline: 12.5802x; 12.5802x over previous
import functools

import jax
import jax.numpy as jnp
from jax import lax
from jax.experimental import pallas as pl
from jax.experimental.pallas import tpu as pltpu
from jax.experimental.pallas import tpu_sc as plsc

M = 6
K = 64
NC = 2
NS = 16
NW = NC * NS
L = 16

CHUNK = 4096
NBUF = 2
UNROLL = 8


@functools.lru_cache(maxsize=None)
def _build(batch: int):
    assert batch % (NW * CHUNK) == 0
    rows_per_worker = batch // NW
    nchunks = rows_per_worker // CHUNK

    mesh = plsc.VectorSubcoreMesh(
        core_axis_name="c", subcore_axis_name="s",
        num_cores=NC, num_subcores=NS,
    )

    @functools.partial(
        pl.kernel,
        out_type=jax.ShapeDtypeStruct((batch,), jnp.float32),
        mesh=mesh,
        scratch_types=[
            pltpu.VMEM((M, CHUNK), jnp.int32),
            pltpu.VMEM((M, CHUNK), jnp.int32),
            pltpu.VMEM((K,), jnp.float32),
            pltpu.VMEM((CHUNK,), jnp.float32),
            pltpu.VMEM((CHUNK,), jnp.float32),
            pltpu.SemaphoreType.DMA,
            pltpu.SemaphoreType.DMA,
            pltpu.SemaphoreType.DMA,
            pltpu.SemaphoreType.DMA,
        ],
        compiler_params=pltpu.CompilerParams(
            needs_layout_passes=False, use_tc_tiling_on_sc=True),
    )
    def mapper(bt_hbm, sym_hbm, out_hbm, b_v0, b_v1, sym_v, o_v0, o_v1,
               si0, si1, so0, so1):
        b_bufs = (b_v0, b_v1)
        o_bufs = (o_v0, o_v1)
        sems_in = (si0, si1)
        sems_out = (so0, so1)

        wid = lax.axis_index("s") * NC + lax.axis_index("c")
        row0 = wid * rows_per_worker
        pltpu.sync_copy(sym_hbm, sym_v)

        def in_copy(c):
            buf = c % NBUF
            return pltpu.make_async_copy(
                bt_hbm.at[:, pl.ds(row0 + c * CHUNK, CHUNK)],
                b_bufs[buf], sems_in[buf])

        def out_copy(c):
            buf = c % NBUF
            return pltpu.make_async_copy(
                o_bufs[buf],
                out_hbm.at[pl.ds(row0 + c * CHUNK, CHUNK)],
                sems_out[buf])

        for c in range(min(NBUF, nchunks)):
            in_copy(c).start()

        for c in range(nchunks):
            buf = c % NBUF
            in_copy(c).wait()
            if c >= NBUF:
                out_copy(c - NBUF).wait()

            bbuf = b_bufs[buf]
            obuf = o_bufs[buf]

            @plsc.parallel_loop(0, CHUNK // L, unroll=UNROLL)
            def _(g):
                off = g * L
                acc = bbuf[0, pl.ds(off, L)]
                for j in range(1, M):
                    acc = acc * 2 + bbuf[j, pl.ds(off, L)]
                obuf[pl.ds(off, L)] = plsc.load_gather(sym_v, [acc])

            out_copy(c).start()
            if c + NBUF < nchunks:
                in_copy(c + NBUF).start()

        for c in range(max(nchunks - NBUF, 0), nchunks):
            out_copy(c).wait()

    return mapper


def kernel(b, symbols):
    batch = b.shape[0]
    flat = _build(batch)(b.T, symbols.reshape(-1))
    return flat.reshape(batch, 1, 1)


# speedup vs baseline: 12.9865x; 1.0323x over previous
import functools

import jax
import jax.numpy as jnp
from jax import lax
from jax.experimental import pallas as pl
from jax.experimental.pallas import tpu as pltpu
from jax.experimental.pallas import tpu_sc as plsc

M = 6
K = 64
NC = 2
NS = 16
NW = NC * NS
L = 16

CHUNK = 4096
NBUF = 3
UNROLL = 16


@functools.lru_cache(maxsize=None)
def _build(batch: int):
    assert batch % (NW * CHUNK) == 0
    rows_per_worker = batch // NW
    nchunks = rows_per_worker // CHUNK

    mesh = plsc.VectorSubcoreMesh(
        core_axis_name="c", subcore_axis_name="s",
        num_cores=NC, num_subcores=NS,
    )

    @functools.partial(
        pl.kernel,
        out_type=jax.ShapeDtypeStruct((batch,), jnp.float32),
        mesh=mesh,
        scratch_types=[
            pltpu.VMEM((M, CHUNK), jnp.int32),
            pltpu.VMEM((M, CHUNK), jnp.int32),
            pltpu.VMEM((M, CHUNK), jnp.int32),
            pltpu.VMEM((K,), jnp.float32),
            pltpu.VMEM((CHUNK,), jnp.float32),
            pltpu.VMEM((CHUNK,), jnp.float32),
            pltpu.VMEM((CHUNK,), jnp.float32),
            pltpu.SemaphoreType.DMA,
            pltpu.SemaphoreType.DMA,
            pltpu.SemaphoreType.DMA,
            pltpu.SemaphoreType.DMA,
            pltpu.SemaphoreType.DMA,
            pltpu.SemaphoreType.DMA,
        ],
        compiler_params=pltpu.CompilerParams(
            needs_layout_passes=False, use_tc_tiling_on_sc=True),
    )
    def mapper(bt_hbm, sym_hbm, out_hbm, b_v0, b_v1, b_v2, sym_v,
               o_v0, o_v1, o_v2, si0, si1, si2, so0, so1, so2):
        b_bufs = (b_v0, b_v1, b_v2)
        o_bufs = (o_v0, o_v1, o_v2)
        sems_in = (si0, si1, si2)
        sems_out = (so0, so1, so2)

        wid = lax.axis_index("s") * NC + lax.axis_index("c")
        row0 = wid * rows_per_worker

        def in_copy(c):
            buf = c % NBUF
            return pltpu.make_async_copy(
                bt_hbm.at[:, pl.ds(row0 + c * CHUNK, CHUNK)],
                b_bufs[buf], sems_in[buf])

        def out_copy(c):
            buf = c % NBUF
            return pltpu.make_async_copy(
                o_bufs[buf],
                out_hbm.at[pl.ds(row0 + c * CHUNK, CHUNK)],
                sems_out[buf])

        for c in range(min(NBUF, nchunks)):
            in_copy(c).start()
        pltpu.sync_copy(sym_hbm, sym_v)

        for c in range(nchunks):
            buf = c % NBUF
            in_copy(c).wait()
            if c >= NBUF:
                out_copy(c - NBUF).wait()

            bbuf = b_bufs[buf]
            obuf = o_bufs[buf]

            @plsc.parallel_loop(0, CHUNK // L, unroll=UNROLL)
            def _(g):
                off = g * L
                b0 = bbuf[0, pl.ds(off, L)]
                b1 = bbuf[1, pl.ds(off, L)]
                b2 = bbuf[2, pl.ds(off, L)]
                b3 = bbuf[3, pl.ds(off, L)]
                b4 = bbuf[4, pl.ds(off, L)]
                b5 = bbuf[5, pl.ds(off, L)]
                p01 = b0 * 2 + b1
                p23 = b2 * 2 + b3
                p45 = b4 * 2 + b5
                acc = (p01 * 4 + p23) * 4 + p45
                obuf[pl.ds(off, L)] = plsc.load_gather(sym_v, [acc])

            out_copy(c).start()
            if c + NBUF < nchunks:
                in_copy(c + NBUF).start()

        for c in range(max(nchunks - NBUF, 0), nchunks):
            out_copy(c).wait()

    return mapper


def kernel(b, symbols):
    batch = b.shape[0]
    flat = _build(batch)(b.T, symbols.reshape(-1))
    return flat.reshape(batch, 1, 1)


# speedup vs baseline: 13.2441x; 1.0198x over previous
import functools

import jax
import jax.numpy as jnp
from jax import lax
from jax.experimental import pallas as pl
from jax.experimental.pallas import tpu as pltpu
from jax.experimental.pallas import tpu_sc as plsc

M = 6
K = 64
NC = 2
NS = 16
NW = NC * NS
L = 16

CHUNK = 4096
NBUF = 3
UNROLL = 8


@functools.lru_cache(maxsize=None)
def _build(batch: int):
    assert batch % (NW * CHUNK) == 0
    rows_per_worker = batch // NW
    nchunks = rows_per_worker // CHUNK

    mesh = plsc.VectorSubcoreMesh(
        core_axis_name="c", subcore_axis_name="s",
        num_cores=NC, num_subcores=NS,
    )

    @functools.partial(
        pl.kernel,
        out_type=jax.ShapeDtypeStruct((batch,), jnp.float32),
        mesh=mesh,
        scratch_types=[
            pltpu.VMEM((M, CHUNK), jnp.int32),
            pltpu.VMEM((M, CHUNK), jnp.int32),
            pltpu.VMEM((M, CHUNK), jnp.int32),
            pltpu.VMEM((K,), jnp.float32),
            pltpu.VMEM((CHUNK,), jnp.float32),
            pltpu.VMEM((CHUNK,), jnp.float32),
            pltpu.VMEM((CHUNK,), jnp.float32),
            pltpu.SemaphoreType.DMA,
            pltpu.SemaphoreType.DMA,
            pltpu.SemaphoreType.DMA,
            pltpu.SemaphoreType.DMA,
            pltpu.SemaphoreType.DMA,
            pltpu.SemaphoreType.DMA,
        ],
        compiler_params=pltpu.CompilerParams(
            needs_layout_passes=False, use_tc_tiling_on_sc=True),
    )
    def mapper(bt_hbm, sym_hbm, out_hbm, b_v0, b_v1, b_v2, sym_v,
               o_v0, o_v1, o_v2, si0, si1, si2, so0, so1, so2):
        b_bufs = (b_v0, b_v1, b_v2)
        o_bufs = (o_v0, o_v1, o_v2)
        sems_in = (si0, si1, si2)
        sems_out = (so0, so1, so2)

        wid = lax.axis_index("s") * NC + lax.axis_index("c")
        row0 = wid * rows_per_worker

        def in_copy(c):
            buf = c % NBUF
            return pltpu.make_async_copy(
                bt_hbm.at[:, pl.ds(row0 + c * CHUNK, CHUNK)],
                b_bufs[buf], sems_in[buf])

        def out_copy(c):
            buf = c % NBUF
            return pltpu.make_async_copy(
                o_bufs[buf],
                out_hbm.at[pl.ds(row0 + c * CHUNK, CHUNK)],
                sems_out[buf])

        for c in range(min(NBUF, nchunks)):
            in_copy(c).start()
        pltpu.sync_copy(sym_hbm, sym_v)

        for c in range(nchunks):
            buf = c % NBUF
            in_copy(c).wait()
            if c >= NBUF:
                out_copy(c - NBUF).wait()

            bbuf = b_bufs[buf]
            obuf = o_bufs[buf]

            @plsc.parallel_loop(0, CHUNK // L, unroll=UNROLL)
            def _(g):
                off = g * L
                b0 = bbuf[0, pl.ds(off, L)]
                b1 = bbuf[1, pl.ds(off, L)]
                b2 = bbuf[2, pl.ds(off, L)]
                b3 = bbuf[3, pl.ds(off, L)]
                b4 = bbuf[4, pl.ds(off, L)]
                b5 = bbuf[5, pl.ds(off, L)]
                p01 = b0 * 2 + b1
                p23 = b2 * 2 + b3
                p45 = b4 * 2 + b5
                acc = (p01 * 4 + p23) * 4 + p45
                obuf[pl.ds(off, L)] = plsc.load_gather(sym_v, [acc])

            out_copy(c).start()
            if c + NBUF < nchunks:
                in_copy(c + NBUF).start()

        for c in range(max(nchunks - NBUF, 0), nchunks):
            out_copy(c).wait()

    return mapper


def kernel(b, symbols):
    batch = b.shape[0]
    flat = _build(batch)(b.T, symbols.reshape(-1))
    return flat.reshape(batch, 1, 1)


# speedup vs baseline: 13.6149x; 1.0280x over previous
import functools

import jax
import jax.numpy as jnp
from jax import lax
from jax.experimental import pallas as pl
from jax.experimental.pallas import tpu as pltpu
from jax.experimental.pallas import tpu_sc as plsc

M = 6
K = 64
NC = 2
NS = 16
NW = NC * NS
L = 16

CHUNK = 4096
NBUF = 3
UNROLL = 4


@functools.lru_cache(maxsize=None)
def _build(batch: int):
    assert batch % (NW * CHUNK) == 0
    rows_per_worker = batch // NW
    nchunks = rows_per_worker // CHUNK

    mesh = plsc.VectorSubcoreMesh(
        core_axis_name="c", subcore_axis_name="s",
        num_cores=NC, num_subcores=NS,
    )

    @functools.partial(
        pl.kernel,
        out_type=jax.ShapeDtypeStruct((batch,), jnp.float32),
        mesh=mesh,
        scratch_types=[
            pltpu.VMEM((M, CHUNK), jnp.int32),
            pltpu.VMEM((M, CHUNK), jnp.int32),
            pltpu.VMEM((M, CHUNK), jnp.int32),
            pltpu.VMEM((K,), jnp.float32),
            pltpu.VMEM((CHUNK,), jnp.float32),
            pltpu.VMEM((CHUNK,), jnp.float32),
            pltpu.VMEM((CHUNK,), jnp.float32),
            pltpu.SemaphoreType.DMA,
            pltpu.SemaphoreType.DMA,
            pltpu.SemaphoreType.DMA,
            pltpu.SemaphoreType.DMA,
            pltpu.SemaphoreType.DMA,
            pltpu.SemaphoreType.DMA,
        ],
        compiler_params=pltpu.CompilerParams(
            needs_layout_passes=False, use_tc_tiling_on_sc=True),
    )
    def mapper(bt_hbm, sym_hbm, out_hbm, b_v0, b_v1, b_v2, sym_v,
               o_v0, o_v1, o_v2, si0, si1, si2, so0, so1, so2):
        b_bufs = (b_v0, b_v1, b_v2)
        o_bufs = (o_v0, o_v1, o_v2)
        sems_in = (si0, si1, si2)
        sems_out = (so0, so1, so2)

        wid = lax.axis_index("s") * NC + lax.axis_index("c")
        row0 = wid * rows_per_worker

        def in_copy(c):
            buf = c % NBUF
            return pltpu.make_async_copy(
                bt_hbm.at[:, pl.ds(row0 + c * CHUNK, CHUNK)],
                b_bufs[buf], sems_in[buf])

        def out_copy(c):
            buf = c % NBUF
            return pltpu.make_async_copy(
                o_bufs[buf],
                out_hbm.at[pl.ds(row0 + c * CHUNK, CHUNK)],
                sems_out[buf])

        for c in range(min(NBUF, nchunks)):
            in_copy(c).start()
        pltpu.sync_copy(sym_hbm, sym_v)

        for c in range(nchunks):
            buf = c % NBUF
            in_copy(c).wait()
            if c >= NBUF:
                out_copy(c - NBUF).wait()

            bbuf = b_bufs[buf]
            obuf = o_bufs[buf]

            @plsc.parallel_loop(0, CHUNK // L, unroll=UNROLL)
            def _(g):
                off = g * L
                b0 = bbuf[0, pl.ds(off, L)]
                b1 = bbuf[1, pl.ds(off, L)]
                b2 = bbuf[2, pl.ds(off, L)]
                b3 = bbuf[3, pl.ds(off, L)]
                b4 = bbuf[4, pl.ds(off, L)]
                b5 = bbuf[5, pl.ds(off, L)]
                p01 = b0 * 2 + b1
                p23 = b2 * 2 + b3
                p45 = b4 * 2 + b5
                acc = (p01 * 4 + p23) * 4 + p45
                obuf[pl.ds(off, L)] = plsc.load_gather(sym_v, [acc])

            out_copy(c).start()
            if c + NBUF < nchunks:
                in_copy(c + NBUF).start()

        for c in range(max(nchunks - NBUF, 0), nchunks):
            out_copy(c).wait()

    return mapper


def kernel(b, symbols):
    batch = b.shape[0]
    flat = _build(batch)(b.T, symbols.reshape(-1))
    return flat.reshape(batch, 1, 1)


# speedup vs baseline: 13.9140x; 1.0220x over previous
import functools

import jax
import jax.numpy as jnp
from jax import lax
from jax.experimental import pallas as pl
from jax.experimental.pallas import tpu as pltpu
from jax.experimental.pallas import tpu_sc as plsc

M = 6
K = 64
NC = 2
NS = 16
NW = NC * NS
L = 16

CHUNK = 4096
NBUF = 3
UNROLL = 2


@functools.lru_cache(maxsize=None)
def _build(batch: int):
    assert batch % (NW * CHUNK) == 0
    rows_per_worker = batch // NW
    nchunks = rows_per_worker // CHUNK

    mesh = plsc.VectorSubcoreMesh(
        core_axis_name="c", subcore_axis_name="s",
        num_cores=NC, num_subcores=NS,
    )

    @functools.partial(
        pl.kernel,
        out_type=jax.ShapeDtypeStruct((batch,), jnp.float32),
        mesh=mesh,
        scratch_types=[
            pltpu.VMEM((M, CHUNK), jnp.int32),
            pltpu.VMEM((M, CHUNK), jnp.int32),
            pltpu.VMEM((M, CHUNK), jnp.int32),
            pltpu.VMEM((K,), jnp.float32),
            pltpu.VMEM((CHUNK,), jnp.float32),
            pltpu.VMEM((CHUNK,), jnp.float32),
            pltpu.VMEM((CHUNK,), jnp.float32),
            pltpu.SemaphoreType.DMA,
            pltpu.SemaphoreType.DMA,
            pltpu.SemaphoreType.DMA,
            pltpu.SemaphoreType.DMA,
            pltpu.SemaphoreType.DMA,
            pltpu.SemaphoreType.DMA,
        ],
        compiler_params=pltpu.CompilerParams(
            needs_layout_passes=False, use_tc_tiling_on_sc=True),
    )
    def mapper(bt_hbm, sym_hbm, out_hbm, b_v0, b_v1, b_v2, sym_v,
               o_v0, o_v1, o_v2, si0, si1, si2, so0, so1, so2):
        b_bufs = (b_v0, b_v1, b_v2)
        o_bufs = (o_v0, o_v1, o_v2)
        sems_in = (si0, si1, si2)
        sems_out = (so0, so1, so2)

        wid = lax.axis_index("s") * NC + lax.axis_index("c")
        row0 = wid * rows_per_worker

        def in_copy(c):
            buf = c % NBUF
            return pltpu.make_async_copy(
                bt_hbm.at[:, pl.ds(row0 + c * CHUNK, CHUNK)],
                b_bufs[buf], sems_in[buf])

        def out_copy(c):
            buf = c % NBUF
            return pltpu.make_async_copy(
                o_bufs[buf],
                out_hbm.at[pl.ds(row0 + c * CHUNK, CHUNK)],
                sems_out[buf])

        for c in range(min(NBUF, nchunks)):
            in_copy(c).start()
        pltpu.sync_copy(sym_hbm, sym_v)

        for c in range(nchunks):
            buf = c % NBUF
            in_copy(c).wait()
            if c >= NBUF:
                out_copy(c - NBUF).wait()

            bbuf = b_bufs[buf]
            obuf = o_bufs[buf]

            @plsc.parallel_loop(0, CHUNK // L, unroll=UNROLL)
            def _(g):
                off = g * L
                b0 = bbuf[0, pl.ds(off, L)]
                b1 = bbuf[1, pl.ds(off, L)]
                b2 = bbuf[2, pl.ds(off, L)]
                b3 = bbuf[3, pl.ds(off, L)]
                b4 = bbuf[4, pl.ds(off, L)]
                b5 = bbuf[5, pl.ds(off, L)]
                p01 = b0 * 2 + b1
                p23 = b2 * 2 + b3
                p45 = b4 * 2 + b5
                acc = (p01 * 4 + p23) * 4 + p45
                obuf[pl.ds(off, L)] = plsc.load_gather(sym_v, [acc])

            out_copy(c).start()
            if c + NBUF < nchunks:
                in_copy(c + NBUF).start()

        for c in range(max(nchunks - NBUF, 0), nchunks):
            out_copy(c).wait()

    return mapper


def kernel(b, symbols):
    batch = b.shape[0]
    flat = _build(batch)(b.T, symbols.reshape(-1))
    return flat.reshape(batch, 1, 1)


# speedup vs baseline: 13.9866x; 1.0052x over previous
import functools

import jax
import jax.numpy as jnp
from jax import lax
from jax.experimental import pallas as pl
from jax.experimental.pallas import tpu as pltpu
from jax.experimental.pallas import tpu_sc as plsc

M = 6
K = 64
NC = 2
NS = 16
NW = NC * NS
L = 16

CHUNK = 4096
NBUF = 3
UNROLL = 1


@functools.lru_cache(maxsize=None)
def _build(batch: int):
    assert batch % (NW * CHUNK) == 0
    rows_per_worker = batch // NW
    nchunks = rows_per_worker // CHUNK

    mesh = plsc.VectorSubcoreMesh(
        core_axis_name="c", subcore_axis_name="s",
        num_cores=NC, num_subcores=NS,
    )

    @functools.partial(
        pl.kernel,
        out_type=jax.ShapeDtypeStruct((batch,), jnp.float32),
        mesh=mesh,
        scratch_types=[
            pltpu.VMEM((M, CHUNK), jnp.int32),
            pltpu.VMEM((M, CHUNK), jnp.int32),
            pltpu.VMEM((M, CHUNK), jnp.int32),
            pltpu.VMEM((K,), jnp.float32),
            pltpu.VMEM((CHUNK,), jnp.float32),
            pltpu.VMEM((CHUNK,), jnp.float32),
            pltpu.VMEM((CHUNK,), jnp.float32),
            pltpu.SemaphoreType.DMA,
            pltpu.SemaphoreType.DMA,
            pltpu.SemaphoreType.DMA,
            pltpu.SemaphoreType.DMA,
            pltpu.SemaphoreType.DMA,
            pltpu.SemaphoreType.DMA,
        ],
        compiler_params=pltpu.CompilerParams(
            needs_layout_passes=False, use_tc_tiling_on_sc=True),
    )
    def mapper(bt_hbm, sym_hbm, out_hbm, b_v0, b_v1, b_v2, sym_v,
               o_v0, o_v1, o_v2, si0, si1, si2, so0, so1, so2):
        b_bufs = (b_v0, b_v1, b_v2)
        o_bufs = (o_v0, o_v1, o_v2)
        sems_in = (si0, si1, si2)
        sems_out = (so0, so1, so2)

        wid = lax.axis_index("s") * NC + lax.axis_index("c")
        row0 = wid * rows_per_worker

        def in_copy(c):
            buf = c % NBUF
            return pltpu.make_async_copy(
                bt_hbm.at[:, pl.ds(row0 + c * CHUNK, CHUNK)],
                b_bufs[buf], sems_in[buf])

        def out_copy(c):
            buf = c % NBUF
            return pltpu.make_async_copy(
                o_bufs[buf],
                out_hbm.at[pl.ds(row0 + c * CHUNK, CHUNK)],
                sems_out[buf])

        for c in range(min(NBUF, nchunks)):
            in_copy(c).start()
        pltpu.sync_copy(sym_hbm, sym_v)

        for c in range(nchunks):
            buf = c % NBUF
            in_copy(c).wait()
            if c >= NBUF:
                out_copy(c - NBUF).wait()

            bbuf = b_bufs[buf]
            obuf = o_bufs[buf]

            @plsc.parallel_loop(0, CHUNK // L, unroll=UNROLL)
            def _(g):
                off = g * L
                b0 = bbuf[0, pl.ds(off, L)]
                b1 = bbuf[1, pl.ds(off, L)]
                b2 = bbuf[2, pl.ds(off, L)]
                b3 = bbuf[3, pl.ds(off, L)]
                b4 = bbuf[4, pl.ds(off, L)]
                b5 = bbuf[5, pl.ds(off, L)]
                p01 = b0 * 2 + b1
                p23 = b2 * 2 + b3
                p45 = b4 * 2 + b5
                acc = (p01 * 4 + p23) * 4 + p45
                obuf[pl.ds(off, L)] = plsc.load_gather(sym_v, [acc])

            out_copy(c).start()
            if c + NBUF < nchunks:
                in_copy(c + NBUF).start()

        for c in range(max(nchunks - NBUF, 0), nchunks):
            out_copy(c).wait()

    return mapper


def kernel(b, symbols):
    batch = b.shape[0]
    flat = _build(batch)(b.T, symbols.reshape(-1))
    return flat.reshape(batch, 1, 1)
